# baseline (speedup 1.0000x reference)
import functools

import jax
import jax.numpy as jnp
from jax import lax
from jax.experimental import pallas as pl
from jax.experimental.pallas import tpu as pltpu
from jax.experimental.pallas import tpu_sc as plsc


_NUM_WORKERS = 32


def _make_sc_gather(F, V, B, D):
    b_per_w = B // _NUM_WORKERS
    mesh = plsc.VectorSubcoreMesh(core_axis_name="c", subcore_axis_name="s")

    @functools.partial(
        pl.kernel,
        mesh=mesh,
        out_type=jax.ShapeDtypeStruct((B, F, D), jnp.float32),
        compiler_params=pltpu.CompilerParams(use_tc_tiling_on_sc=False),
        scratch_types=[
            pltpu.VMEM((F, b_per_w), jnp.int32),
            pltpu.VMEM((b_per_w, D), jnp.float32),
            pltpu.SemaphoreType.DMA,
        ],
    )
    def gather_k(table_hbm, ls_hbm, out_hbm, lsv, rows_v, sem):
        wid = lax.axis_index("s") * 2 + lax.axis_index("c")
        b0 = wid * b_per_w
        pltpu.sync_copy(ls_hbm.at[:, pl.ds(b0, b_per_w)], lsv)
        for f in range(F):
            pltpu.async_copy(table_hbm.at[f].at[lsv.at[f]], rows_v, sem).wait()
            pltpu.sync_copy(rows_v, out_hbm.at[pl.ds(b0, b_per_w), f])

    return gather_k


_BLK = 2048


def _mlp_body(dx, lyb, tg, wb1, bb1, wb2, bb2, w1a, w1b, bt1, wt2, bt2, out):
    i = pl.program_id(0)
    f32 = jnp.float32
    x = jnp.dot(dx[...], wb1[...], preferred_element_type=f32) + bb1[...]
    x = jnp.dot(x, wb2[...], preferred_element_type=f32) + bb2[...]
    x = jnp.maximum(x, 0.0)
    h = (
        jnp.dot(x, w1a[...], preferred_element_type=f32)
        + jnp.dot(lyb[...], w1b[...], preferred_element_type=f32)
        + bt1[...]
    )
    s = jnp.dot(h, wt2[...], preferred_element_type=f32) + bt2[...]
    p = jax.nn.sigmoid(s)
    t = tg[...]
    log_p = jnp.maximum(jnp.log(p), -100.0)
    log_1mp = jnp.maximum(jnp.log(1.0 - p), -100.0)
    blk_sum = jnp.sum(t * log_p + (1.0 - t) * log_1mp)

    @pl.when(i == 0)
    def _():
        out[0, 0] = 0.0

    out[0, 0] += blk_sum


def _mlp_loss(dense_x, ly, target, W_bot1, b_bot1, W_bot2, b_bot2,
              W1a, W1b, b_top1, W_top2, b_top2):
    B = dense_x.shape[0]
    FD = ly.shape[1]
    grid = (B // _BLK,)
    full = lambda shape: pl.BlockSpec(shape, lambda i: (0, 0))
    out = pl.pallas_call(
        _mlp_body,
        grid=grid,
        in_specs=[
            pl.BlockSpec((_BLK, dense_x.shape[1]), lambda i: (i, 0)),
            pl.BlockSpec((_BLK, FD), lambda i: (i, 0)),
            pl.BlockSpec((_BLK, 1), lambda i: (i, 0)),
            full(W_bot1.shape),
            full(b_bot1.shape),
            full(W_bot2.shape),
            full(b_bot2.shape),
            full(W1a.shape),
            full(W1b.shape),
            full(b_top1.shape),
            full(W_top2.shape),
            full(b_top2.shape),
        ],
        out_specs=pl.BlockSpec(memory_space=pltpu.SMEM),
        out_shape=jax.ShapeDtypeStruct((1, 1), jnp.float32),
    )(dense_x, ly, target, W_bot1, b_bot1, W_bot2, b_bot2,
      W1a, W1b, b_top1, W_top2, b_top2)
    return out


def kernel(dense_x, ls_i, target, W_bot1, b_bot1, W_bot2, b_bot2, emb,
           W_top1, b_top1, W_top2, b_top2):
    F, V, D = emb.shape
    B = dense_x.shape[0]
    N = F * B

    rows = _make_sc_gather(F, V, B, D)(emb, ls_i)
    ly = rows.reshape(B, F * D)

    loss_sum = _mlp_loss(
        dense_x, ly, target,
        W_bot1, b_bot1.reshape(1, -1), W_bot2, b_bot2.reshape(1, -1),
        W_top1[:D], W_top1[D:], b_top1.reshape(1, -1),
        W_top2, b_top2.reshape(1, 1),
    )
    return -loss_sum[0, 0] / B


# speedup vs baseline: 1.1538x; 1.1538x over previous
import functools

import jax
import jax.numpy as jnp
from jax import lax
from jax.experimental import pallas as pl
from jax.experimental.pallas import tpu as pltpu
from jax.experimental.pallas import tpu_sc as plsc

_NUM_WORKERS = 32
_VCH = 1024
_PAD = 128


def _tr_body(in_ref, out_ref):
    x = in_ref[0]
    out_ref[0, :, : x.shape[0]] = x.T
    out_ref[0, :, x.shape[0] :] = jnp.zeros(
        (x.shape[1], _PAD - x.shape[0]), jnp.float32
    )


def _transpose_table(emb_T):
    F, D, V = emb_T.shape
    grid = (F, pl.cdiv(V, _VCH))
    return pl.pallas_call(
        _tr_body,
        grid=grid,
        in_specs=[pl.BlockSpec((1, D, _VCH), lambda f, c: (f, 0, c))],
        out_specs=pl.BlockSpec((1, _VCH, _PAD), lambda f, c: (f, c, 0)),
        out_shape=jax.ShapeDtypeStruct((F, V, _PAD), jnp.float32),
    )(emb_T)


def _make_sc_gather(F, V, B):
    b_per_w = B // _NUM_WORKERS
    mesh = plsc.VectorSubcoreMesh(core_axis_name="c", subcore_axis_name="s")

    @functools.partial(
        pl.kernel,
        mesh=mesh,
        out_type=jax.ShapeDtypeStruct((F, B, _PAD), jnp.float32),
        compiler_params=pltpu.CompilerParams(use_tc_tiling_on_sc=True),
        scratch_types=[
            pltpu.VMEM((b_per_w,), jnp.int32),
            pltpu.VMEM((b_per_w, _PAD), jnp.float32),
            pltpu.SemaphoreType.DMA,
        ],
    )
    def gather_k(table_hbm, ls_hbm, out_hbm, idx_v, rows_v, sem):
        wid = lax.axis_index("s") * 2 + lax.axis_index("c")
        b0 = wid * b_per_w
        for f in range(F):
            pltpu.sync_copy(ls_hbm.at[f].at[pl.ds(b0, b_per_w)], idx_v)
            pltpu.async_copy(table_hbm.at[f].at[idx_v], rows_v, sem).wait()
            pltpu.sync_copy(rows_v, out_hbm.at[f].at[pl.ds(b0, b_per_w)])

    return gather_k


_BLK = 1024


def _mlp_body(dx, lyb, tg, wb1, bb1, wb2, bb2, w1a, w1b, bt1, wt2, bt2, out):
    i = pl.program_id(0)
    F = lyb.shape[0]
    f32 = jnp.float32
    x = jnp.dot(dx[...], wb1[...], preferred_element_type=f32) + bb1[...]
    x = jnp.dot(x, wb2[...], preferred_element_type=f32) + bb2[...]
    x = jnp.maximum(x, 0.0)
    h = jnp.dot(x, w1a[...], preferred_element_type=f32) + bt1[...]
    for f in range(F):
        h += jnp.dot(lyb[f], w1b[f], preferred_element_type=f32)
    s = jnp.dot(h, wt2[...], preferred_element_type=f32) + bt2[...]
    p = jax.nn.sigmoid(s)
    t = tg[...]
    log_p = jnp.maximum(jnp.log(p), -100.0)
    log_1mp = jnp.maximum(jnp.log(1.0 - p), -100.0)
    blk_sum = jnp.sum(t * log_p + (1.0 - t) * log_1mp)

    @pl.when(i == 0)
    def _():
        out[0, 0] = 0.0

    out[0, 0] += blk_sum


def _mlp_loss(dense_x, ly, target, W_bot1, b_bot1, W_bot2, b_bot2,
              W1a, W1b_pad, b_top1, W_top2, b_top2):
    B = dense_x.shape[0]
    F = ly.shape[0]
    grid = (B // _BLK,)
    full = lambda a: pl.BlockSpec(a.shape, lambda i: (0,) * a.ndim)
    out = pl.pallas_call(
        _mlp_body,
        grid=grid,
        in_specs=[
            pl.BlockSpec((_BLK, dense_x.shape[1]), lambda i: (i, 0)),
            pl.BlockSpec((F, _BLK, _PAD), lambda i: (0, i, 0)),
            pl.BlockSpec((_BLK, 1), lambda i: (i, 0)),
            full(W_bot1),
            full(b_bot1),
            full(W_bot2),
            full(b_bot2),
            full(W1a),
            full(W1b_pad),
            full(b_top1),
            full(W_top2),
            full(b_top2),
        ],
        out_specs=pl.BlockSpec(memory_space=pltpu.SMEM),
        out_shape=jax.ShapeDtypeStruct((1, 1), jnp.float32),
    )(dense_x, ly, target, W_bot1, b_bot1, W_bot2, b_bot2,
      W1a, W1b_pad, b_top1, W_top2, b_top2)
    return out


def kernel(dense_x, ls_i, target, W_bot1, b_bot1, W_bot2, b_bot2, emb,
           W_top1, b_top1, W_top2, b_top2):
    F, V, D = emb.shape
    B = dense_x.shape[0]

    emb_T = jnp.swapaxes(emb, 1, 2)
    table = _transpose_table(emb_T)
    ly = _make_sc_gather(F, V, B)(table, ls_i)

    W1b_pad = jnp.pad(
        W_top1[D:].reshape(F, D, -1), ((0, 0), (0, _PAD - D), (0, 0))
    )
    loss_sum = _mlp_loss(
        dense_x, ly, target,
        W_bot1, b_bot1.reshape(1, -1), W_bot2, b_bot2.reshape(1, -1),
        W_top1[:D], W1b_pad, b_top1.reshape(1, -1),
        W_top2, b_top2.reshape(1, 1),
    )
    return -loss_sum[0, 0] / B
